# Initial kernel scaffold; baseline (speedup 1.0000x reference)
import jax
import jax.numpy as jnp
from jax.experimental import pallas as pl


def kernel(x, edge_index, edge_weight):
    raise NotImplementedError("write your pallas kernel here")


# speedup vs baseline: 4.0785x; 4.0785x over previous
import jax
import jax.numpy as jnp
from jax import lax
from jax.experimental import pallas as pl
from jax.experimental.pallas import tpu as pltpu
from jax.experimental.pallas import tpu_sc as plsc

N_NODES = 10000
N_EDGES = 320000
D_FEAT = 128
ITERS = 10
ALPHA = 0.5

NC = 2
NS = 16
PASSES = 2
COLS = D_FEAT // (NC * PASSES)
VPR = COLS // 16
BLK = 128
EPT = -(-N_EDGES // NS)
NBLK = -(-EPT // BLK)
EPT_PAD = NBLK * BLK
ROWS = 640
N_PAD = NS * ROWS
RCHUNK = 128
NCHUNK = ROWS // RCHUNK


def _body(x_hbm, src_hbm, dst_hbm, w_hbm, out_hbm,
          src_ref, dst_ref, w_ref, gbuf, di_ref, ta_sp, tb_sp, sem):
    c = lax.axis_index("c")
    s = lax.axis_index("s")
    row0 = s * ROWS

    pltpu.sync_copy(src_hbm.at[s], src_ref)
    pltpu.sync_copy(dst_hbm.at[s], dst_ref)
    pltpu.sync_copy(w_hbm.at[s], w_ref)
    def wscale(i, carry):
        for g in range(BLK // 16):
            sl = pl.ds(g * 16, 16)
            w_ref[i, sl] = w_ref[i, sl] * ALPHA
        return carry
    lax.fori_loop(0, NBLK, wscale, 0)

    for p in range(PASSES):
        xp = x_hbm.at[PASSES * c + p]
        op = out_hbm.at[PASSES * c + p]

        for ch in range(NCHUNK):
            r = row0 + ch * RCHUNK
            pltpu.sync_copy(xp.at[pl.ds(r, RCHUNK)], gbuf)

            def init_row(r2, carry, ch=ch):
                for g in range(VPR):
                    sl = pl.ds(g * 16, 16)
                    v = gbuf[r2, sl]
                    di_ref[ch * RCHUNK + r2, sl] = v * ((1.0 - ALPHA) / ALPHA)
                    gbuf[r2, sl] = v * (1.0 / ALPHA)
                return carry
            lax.fori_loop(0, RCHUNK, init_row, 0)
            pltpu.sync_copy(gbuf, ta_sp.at[pl.ds(r, RCHUNK)])
        pltpu.sync_copy(di_ref, tb_sp.at[pl.ds(row0, ROWS)])
        plsc.subcore_barrier()

        for it in range(ITERS):
            tsrc = ta_sp if it % 2 == 0 else tb_sp
            tdst = tb_sp if it % 2 == 0 else ta_sp

            def blk_body(b, carry):
                pltpu.async_copy(tsrc.at[src_ref.at[b]], gbuf, sem).wait()

                def edge_body(e, carry2):
                    bi = jnp.full((16,), b, jnp.int32)
                    ei = jnp.full((16,), e, jnp.int32)
                    wv = plsc.load_gather(w_ref, [bi, ei])
                    for g in range(VPR):
                        sl = pl.ds(g * 16, 16)
                        gbuf[e, sl] = gbuf[e, sl] * wv
                    return carry2
                lax.fori_loop(0, BLK, edge_body, 0)

                pltpu.sync_copy(gbuf, tdst.at[dst_ref.at[b]], add=True)
                return carry
            lax.fori_loop(0, NBLK, blk_body, 0)
            plsc.subcore_barrier()

            if it < ITERS - 1:
                pltpu.sync_copy(di_ref, tsrc.at[pl.ds(row0, ROWS)])
                plsc.subcore_barrier()

        tlast = tb_sp if (ITERS - 1) % 2 == 0 else ta_sp
        for ch in range(NCHUNK):
            r = row0 + ch * RCHUNK
            pltpu.sync_copy(tlast.at[pl.ds(r, RCHUNK)], gbuf)

            def out_row(r2, carry):
                for g in range(VPR):
                    sl = pl.ds(g * 16, 16)
                    gbuf[r2, sl] = gbuf[r2, sl] * ALPHA
                return carry
            lax.fori_loop(0, RCHUNK, out_row, 0)
            pltpu.sync_copy(gbuf, op.at[pl.ds(r, RCHUNK)])
        if p != PASSES - 1:
            plsc.subcore_barrier()


@jax.jit
def _run(x_split, src, dst, w):
    mesh = plsc.VectorSubcoreMesh(core_axis_name="c", subcore_axis_name="s")
    return pl.kernel(
        _body,
        out_type=jax.ShapeDtypeStruct((NC * PASSES, N_PAD, COLS), jnp.float32),
        mesh=mesh,
        scratch_types=[
            pltpu.VMEM((NBLK, BLK), jnp.int32),
            pltpu.VMEM((NBLK, BLK), jnp.int32),
            pltpu.VMEM((NBLK, BLK), jnp.float32),
            pltpu.VMEM((RCHUNK, COLS), jnp.float32),
            pltpu.VMEM((ROWS, COLS), jnp.float32),
            pltpu.VMEM_SHARED((N_PAD, COLS), jnp.float32),
            pltpu.VMEM_SHARED((N_PAD, COLS), jnp.float32),
            pltpu.SemaphoreType.DMA,
        ],
        compiler_params=pltpu.CompilerParams(needs_layout_passes=False, use_tc_tiling_on_sc=False),
    )(x_split, src, dst, w)


def kernel(x, edge_index, edge_weight):
    dst = edge_index[0].astype(jnp.int32)
    src = edge_index[1].astype(jnp.int32)
    pad = NS * EPT_PAD - N_EDGES
    src_p = jnp.concatenate([src, jnp.zeros((pad,), jnp.int32)]).reshape(NS, NBLK, BLK)
    dst_p = jnp.concatenate([dst, jnp.zeros((pad,), jnp.int32)]).reshape(NS, NBLK, BLK)
    w_p = jnp.concatenate(
        [edge_weight, jnp.zeros((pad,), jnp.float32)]).reshape(NS, NBLK, BLK)
    x_pad = jnp.pad(x, ((0, N_PAD - N_NODES), (0, 0)))
    x_split = jnp.moveaxis(x_pad.reshape(N_PAD, NC * PASSES, COLS), 1, 0)
    out_split = _run(x_split, src_p, dst_p, w_p)
    return jnp.moveaxis(out_split, 0, 1).reshape(N_PAD, D_FEAT)[:N_NODES]


# speedup vs baseline: 11.3791x; 2.7900x over previous
import jax
import jax.numpy as jnp
from jax import lax
from jax.experimental import pallas as pl
from jax.experimental.pallas import tpu as pltpu
from jax.experimental.pallas import tpu_sc as plsc

N_NODES = 10000
N_EDGES = 320000
D_FEAT = 128
ITERS = 10
ALPHA = 0.5

NC = 2
NS = 16
COLS = D_FEAT // NC
VPR = COLS // 16
BLK = 128
GRP = 16
EPT = -(-N_EDGES // NS)
NGRP = -(-EPT // (GRP * BLK))
NBLK = NGRP * GRP
EPT_PAD = NBLK * BLK
ROWS = 640
N_PAD = NS * ROWS
RCHUNK = 128
NCHUNK = ROWS // RCHUNK


def _body(x_hbm, si_hbm, w_hbm, out_hbm,
          eb, wb, g0, g1, s0, s1, ta_sp, tb_sp,
          gs0, gs1, ss0, ss1, es):
    c = lax.axis_index("c")
    s = lax.axis_index("s")
    row0 = s * ROWS

    xp = x_hbm.at[c]
    op = out_hbm.at[c]
    sip = si_hbm.at[s]
    wp = w_hbm.at[s]

    def estart(m):
        r = lax.rem(m, 2)
        pltpu.async_copy(sip.at[pl.ds(m * 2 * GRP, 2 * GRP)],
                         eb.at[pl.ds(r * 2 * GRP, 2 * GRP)], es)
        pltpu.async_copy(wp.at[pl.ds(m * GRP, GRP)],
                         wb.at[pl.ds(r * GRP, GRP)], es)

    def ewait(m):
        r = lax.rem(m, 2)
        pltpu.make_async_copy(sip.at[pl.ds(m * 2 * GRP, 2 * GRP)],
                              eb.at[pl.ds(r * 2 * GRP, 2 * GRP)], es).wait()
        pltpu.make_async_copy(wp.at[pl.ds(m * GRP, GRP)],
                              wb.at[pl.ds(r * GRP, GRP)], es).wait()

    def srow(b):
        return lax.rem(lax.div(b, GRP), 2) * 2 * GRP + lax.rem(b, GRP) * 2

    def wrow(b):
        return lax.rem(lax.div(b, GRP), 2) * GRP + lax.rem(b, GRP)

    def mult(b, src_buf, dst_buf):
        wr = wrow(b)

        def grp_body(q, carry):
            w16 = wb[wr, pl.ds(q * 16, 16)] * ALPHA
            for k in range(16):
                e = q * 16 + k
                wv = jnp.broadcast_to(w16[k], (16,))
                for g in range(VPR):
                    sl = pl.ds(g * 16, 16)
                    dst_buf[e, sl] = src_buf[e, sl] * wv
            return carry
        lax.fori_loop(0, BLK // 16, grp_body, 0)

    for ch in range(NCHUNK):
        r = row0 + ch * RCHUNK
        pltpu.sync_copy(xp.at[pl.ds(r, RCHUNK)], g0)

        def init_row(r2, carry):
            for g in range(VPR):
                sl = pl.ds(g * 16, 16)
                v = g0[r2, sl]
                g1[r2, sl] = v * ((1.0 - ALPHA) / ALPHA)
                g0[r2, sl] = v * (1.0 / ALPHA)
            return carry
        lax.fori_loop(0, RCHUNK, init_row, 0)
        pltpu.sync_copy(g0, ta_sp.at[pl.ds(r, RCHUNK)])
        pltpu.sync_copy(g1, tb_sp.at[pl.ds(r, RCHUNK)])
        pltpu.sync_copy(g1, op.at[pl.ds(r, RCHUNK)])
    plsc.subcore_barrier()

    def do_iter(tsrc, tdst):
        def gstart(b, buf, sem):
            pltpu.async_copy(tsrc.at[eb.at[srow(b)]], buf, sem)

        def gwait(b, buf, sem):
            pltpu.make_async_copy(tsrc.at[eb.at[srow(b)]], buf, sem).wait()

        def sstart(b, buf, sem):
            pltpu.async_copy(buf, tdst.at[eb.at[srow(b) + 1]], sem, add=True)

        def swait(b, buf, sem):
            pltpu.make_async_copy(buf, tdst.at[eb.at[srow(b) + 1]], sem).wait()

        estart(0)
        ewait(0)
        gstart(0, g0, gs0)
        gstart(1, g1, gs1)
        gwait(0, g0, gs0)
        mult(0, g0, s0)
        gstart(2, g0, gs0)
        sstart(0, s0, ss0)
        gwait(1, g1, gs1)
        mult(1, g1, s1)
        gstart(3, g1, gs1)
        sstart(1, s1, ss1)

        def steady(g, carry):
            b2 = 2 * g
            b3 = 2 * g + 1
            m = lax.div(b2, GRP)
            bm = lax.rem(b2, GRP)

            @pl.when(jnp.logical_and(bm == 2, m < NGRP - 1))
            def _():
                estart(m + 1)

            @pl.when(jnp.logical_and(bm == GRP - 2, m < NGRP - 1))
            def _():
                ewait(m + 1)

            gwait(b2, g0, gs0)
            swait(b2 - 2, s0, ss0)
            mult(b2, g0, s0)
            gstart(b2 + 2, g0, gs0)
            sstart(b2, s0, ss0)
            gwait(b3, g1, gs1)
            swait(b3 - 2, s1, ss1)
            mult(b3, g1, s1)
            gstart(b3 + 2, g1, gs1)
            sstart(b3, s1, ss1)
            return carry
        lax.fori_loop(1, NBLK // 2 - 1, steady, 0)

        b2 = NBLK - 2
        b3 = NBLK - 1
        gwait(b2, g0, gs0)
        swait(b2 - 2, s0, ss0)
        mult(b2, g0, s0)
        sstart(b2, s0, ss0)
        gwait(b3, g1, gs1)
        swait(b3 - 2, s1, ss1)
        mult(b3, g1, s1)
        sstart(b3, s1, ss1)
        swait(b2, s0, ss0)
        swait(b3, s1, ss1)
        plsc.subcore_barrier()

    def two_iters(k, carry):
        do_iter(ta_sp, tb_sp)
        pltpu.sync_copy(op.at[pl.ds(row0, ROWS)], ta_sp.at[pl.ds(row0, ROWS)])
        plsc.subcore_barrier()
        do_iter(tb_sp, ta_sp)
        pltpu.sync_copy(op.at[pl.ds(row0, ROWS)], tb_sp.at[pl.ds(row0, ROWS)])
        plsc.subcore_barrier()
        return carry
    lax.fori_loop(0, ITERS // 2, two_iters, 0)

    for ch in range(NCHUNK):
        r = row0 + ch * RCHUNK
        pltpu.sync_copy(ta_sp.at[pl.ds(r, RCHUNK)], g0)

        def out_row(r2, carry):
            for g in range(VPR):
                sl = pl.ds(g * 16, 16)
                g0[r2, sl] = g0[r2, sl] * ALPHA
            return carry
        lax.fori_loop(0, RCHUNK, out_row, 0)
        pltpu.sync_copy(g0, op.at[pl.ds(r, RCHUNK)])


@jax.jit
def _run(x_split, si, w):
    mesh = plsc.VectorSubcoreMesh(core_axis_name="c", subcore_axis_name="s")
    return pl.kernel(
        _body,
        out_type=jax.ShapeDtypeStruct((NC, N_PAD, COLS), jnp.float32),
        mesh=mesh,
        scratch_types=[
            pltpu.VMEM((2 * 2 * GRP, BLK), jnp.int32),
            pltpu.VMEM((2 * GRP, BLK), jnp.float32),
            pltpu.VMEM((BLK, COLS), jnp.float32),
            pltpu.VMEM((BLK, COLS), jnp.float32),
            pltpu.VMEM((BLK, COLS), jnp.float32),
            pltpu.VMEM((BLK, COLS), jnp.float32),
            pltpu.VMEM_SHARED((N_PAD, COLS), jnp.float32),
            pltpu.VMEM_SHARED((N_PAD, COLS), jnp.float32),
            pltpu.SemaphoreType.DMA,
            pltpu.SemaphoreType.DMA,
            pltpu.SemaphoreType.DMA,
            pltpu.SemaphoreType.DMA,
            pltpu.SemaphoreType.DMA,
        ],
        compiler_params=pltpu.CompilerParams(
            needs_layout_passes=False, use_tc_tiling_on_sc=False),
    )(x_split, si, w)


def kernel(x, edge_index, edge_weight):
    dst = edge_index[0].astype(jnp.int32)
    src = edge_index[1].astype(jnp.int32)
    pad = NS * EPT_PAD - N_EDGES
    src_p = jnp.concatenate([src, jnp.zeros((pad,), jnp.int32)]).reshape(NS, NBLK, BLK)
    dst_p = jnp.concatenate([dst, jnp.zeros((pad,), jnp.int32)]).reshape(NS, NBLK, BLK)
    si = jnp.stack([src_p, dst_p], axis=2).reshape(NS, NBLK * 2, BLK)
    w_p = jnp.concatenate(
        [edge_weight, jnp.zeros((pad,), jnp.float32)]).reshape(NS, NBLK, BLK)
    x_pad = jnp.pad(x, ((0, N_PAD - N_NODES), (0, 0)))
    x_split = jnp.moveaxis(x_pad.reshape(N_PAD, NC, COLS), 1, 0)
    out_split = _run(x_split, si, w_p)
    return jnp.moveaxis(out_split, 0, 1).reshape(N_PAD, D_FEAT)[:N_NODES]


# speedup vs baseline: 11.4125x; 1.0029x over previous
import jax
import jax.numpy as jnp
from jax import lax
from jax.experimental import pallas as pl
from jax.experimental.pallas import tpu as pltpu
from jax.experimental.pallas import tpu_sc as plsc

N_NODES = 10000
N_EDGES = 320000
D_FEAT = 128
ITERS = 10
ALPHA = 0.5

NC = 2
NS = 16
COLS = D_FEAT // NC
VPR = COLS // 16
BLK = 128
GRP = 16
EPT = -(-N_EDGES // NS)
NGRP = -(-EPT // (GRP * BLK))
NBLK = NGRP * GRP
EPT_PAD = NBLK * BLK
ROWS = 640
N_PAD = NS * ROWS
RCHUNK = 128
NCHUNK = ROWS // RCHUNK


def _body(x_hbm, si_hbm, w_hbm, out_hbm,
          eb, wb, g0, g1, s0, s1, ta_sp, tb_sp,
          gs0, gs1, ss0, ss1, es):
    c = lax.axis_index("c")
    s = lax.axis_index("s")
    row0 = s * ROWS

    xp = x_hbm.at[c]
    op = out_hbm.at[c]
    sip = si_hbm.at[s]
    wp = w_hbm.at[s]

    def estart(m):
        r = lax.rem(m, 2)
        pltpu.async_copy(sip.at[pl.ds(m * 2 * GRP, 2 * GRP)],
                         eb.at[pl.ds(r * 2 * GRP, 2 * GRP)], es)
        pltpu.async_copy(wp.at[pl.ds(m * GRP, GRP)],
                         wb.at[pl.ds(r * GRP, GRP)], es)

    def ewait(m):
        r = lax.rem(m, 2)
        pltpu.make_async_copy(sip.at[pl.ds(m * 2 * GRP, 2 * GRP)],
                              eb.at[pl.ds(r * 2 * GRP, 2 * GRP)], es).wait()
        pltpu.make_async_copy(wp.at[pl.ds(m * GRP, GRP)],
                              wb.at[pl.ds(r * GRP, GRP)], es).wait()

    def srow(b):
        return lax.rem(lax.div(b, GRP), 2) * 2 * GRP + lax.rem(b, GRP) * 2

    def wrow(b):
        return lax.rem(lax.div(b, GRP), 2) * GRP + lax.rem(b, GRP)

    def mult(b, src_buf, dst_buf):
        wr = wrow(b)

        def grp_body(q, carry):
            w16 = wb[wr, pl.ds(q * 16, 16)] * ALPHA
            for k in range(16):
                e = q * 16 + k
                wv = jnp.broadcast_to(w16[k], (16,))
                for g in range(VPR):
                    sl = pl.ds(g * 16, 16)
                    dst_buf[e, sl] = src_buf[e, sl] * wv
            return carry
        lax.fori_loop(0, BLK // 16, grp_body, 0)

    for ch in range(NCHUNK):
        r = row0 + ch * RCHUNK
        pltpu.sync_copy(xp.at[pl.ds(r, RCHUNK)], g0)

        def init_row(r2, carry):
            for g in range(VPR):
                sl = pl.ds(g * 16, 16)
                v = g0[r2, sl]
                g1[r2, sl] = v * ((1.0 - ALPHA) / ALPHA)
                g0[r2, sl] = v * (1.0 / ALPHA)
            return carry
        lax.fori_loop(0, RCHUNK, init_row, 0)
        pltpu.sync_copy(g0, ta_sp.at[pl.ds(r, RCHUNK)])
        pltpu.sync_copy(g1, tb_sp.at[pl.ds(r, RCHUNK)])
        pltpu.sync_copy(g1, op.at[pl.ds(r, RCHUNK)])
    estart(0)
    plsc.subcore_barrier()

    def do_iter(tsrc, tdst):
        def gstart(b, buf, sem):
            pltpu.async_copy(tsrc.at[eb.at[srow(b)]], buf, sem)

        def gwait(b, buf, sem):
            pltpu.make_async_copy(tsrc.at[eb.at[srow(b)]], buf, sem).wait()

        def sstart(b, buf, sem):
            pltpu.async_copy(buf, tdst.at[eb.at[srow(b) + 1]], sem, add=True)

        def swait(b, buf, sem):
            pltpu.make_async_copy(buf, tdst.at[eb.at[srow(b) + 1]], sem).wait()

        ewait(0)
        gstart(0, g0, gs0)
        gstart(1, g1, gs1)
        gwait(0, g0, gs0)
        mult(0, g0, s0)
        gstart(2, g0, gs0)
        sstart(0, s0, ss0)
        gwait(1, g1, gs1)
        mult(1, g1, s1)
        gstart(3, g1, gs1)
        sstart(1, s1, ss1)

        def steady(g, carry):
            b2 = 2 * g
            b3 = 2 * g + 1
            m = lax.div(b2, GRP)
            bm = lax.rem(b2, GRP)

            @pl.when(bm == 2)
            def _():
                estart(lax.rem(m + 1, NGRP))

            @pl.when(jnp.logical_and(bm == GRP - 2, m < NGRP - 1))
            def _():
                ewait(m + 1)

            gwait(b2, g0, gs0)
            swait(b2 - 2, s0, ss0)
            mult(b2, g0, s0)
            gstart(b2 + 2, g0, gs0)
            sstart(b2, s0, ss0)
            gwait(b3, g1, gs1)
            swait(b3 - 2, s1, ss1)
            mult(b3, g1, s1)
            gstart(b3 + 2, g1, gs1)
            sstart(b3, s1, ss1)
            return carry
        lax.fori_loop(1, NBLK // 2 - 1, steady, 0)

        b2 = NBLK - 2
        b3 = NBLK - 1
        gwait(b2, g0, gs0)
        swait(b2 - 2, s0, ss0)
        mult(b2, g0, s0)
        sstart(b2, s0, ss0)
        gwait(b3, g1, gs1)
        swait(b3 - 2, s1, ss1)
        mult(b3, g1, s1)
        sstart(b3, s1, ss1)
        swait(b2, s0, ss0)
        swait(b3, s1, ss1)
        plsc.subcore_barrier()

    def two_iters(k, carry):
        do_iter(ta_sp, tb_sp)
        pltpu.sync_copy(op.at[pl.ds(row0, ROWS)], ta_sp.at[pl.ds(row0, ROWS)])
        plsc.subcore_barrier()
        do_iter(tb_sp, ta_sp)
        pltpu.sync_copy(op.at[pl.ds(row0, ROWS)], tb_sp.at[pl.ds(row0, ROWS)])
        plsc.subcore_barrier()
        return carry
    lax.fori_loop(0, ITERS // 2, two_iters, 0)
    ewait(0)

    for ch in range(NCHUNK):
        r = row0 + ch * RCHUNK
        pltpu.sync_copy(ta_sp.at[pl.ds(r, RCHUNK)], g0)

        def out_row(r2, carry):
            for g in range(VPR):
                sl = pl.ds(g * 16, 16)
                g0[r2, sl] = g0[r2, sl] * ALPHA
            return carry
        lax.fori_loop(0, RCHUNK, out_row, 0)
        pltpu.sync_copy(g0, op.at[pl.ds(r, RCHUNK)])


@jax.jit
def _run(x_split, si, w):
    mesh = plsc.VectorSubcoreMesh(core_axis_name="c", subcore_axis_name="s")
    return pl.kernel(
        _body,
        out_type=jax.ShapeDtypeStruct((NC, N_PAD, COLS), jnp.float32),
        mesh=mesh,
        scratch_types=[
            pltpu.VMEM((2 * 2 * GRP, BLK), jnp.int32),
            pltpu.VMEM((2 * GRP, BLK), jnp.float32),
            pltpu.VMEM((BLK, COLS), jnp.float32),
            pltpu.VMEM((BLK, COLS), jnp.float32),
            pltpu.VMEM((BLK, COLS), jnp.float32),
            pltpu.VMEM((BLK, COLS), jnp.float32),
            pltpu.VMEM_SHARED((N_PAD, COLS), jnp.float32),
            pltpu.VMEM_SHARED((N_PAD, COLS), jnp.float32),
            pltpu.SemaphoreType.DMA,
            pltpu.SemaphoreType.DMA,
            pltpu.SemaphoreType.DMA,
            pltpu.SemaphoreType.DMA,
            pltpu.SemaphoreType.DMA,
        ],
        compiler_params=pltpu.CompilerParams(
            needs_layout_passes=False, use_tc_tiling_on_sc=False),
    )(x_split, si, w)


def kernel(x, edge_index, edge_weight):
    dst = edge_index[0].astype(jnp.int32)
    src = edge_index[1].astype(jnp.int32)
    pad = NS * EPT_PAD - N_EDGES
    src_p = jnp.concatenate([src, jnp.zeros((pad,), jnp.int32)]).reshape(NS, NBLK, BLK)
    dst_p = jnp.concatenate([dst, jnp.zeros((pad,), jnp.int32)]).reshape(NS, NBLK, BLK)
    si = jnp.stack([src_p, dst_p], axis=2).reshape(NS, NBLK * 2, BLK)
    w_p = jnp.concatenate(
        [edge_weight, jnp.zeros((pad,), jnp.float32)]).reshape(NS, NBLK, BLK)
    x_pad = jnp.pad(x, ((0, N_PAD - N_NODES), (0, 0)))
    x_split = jnp.moveaxis(x_pad.reshape(N_PAD, NC, COLS), 1, 0)
    out_split = _run(x_split, si, w_p)
    return jnp.moveaxis(out_split, 0, 1).reshape(N_PAD, D_FEAT)[:N_NODES]
